# Initial kernel scaffold; baseline (speedup 1.0000x reference)
import jax
import jax.numpy as jnp
from jax.experimental import pallas as pl


def kernel(hidden_states, W, b):
    raise NotImplementedError("write your pallas kernel here")


# speedup vs baseline: 3.1805x; 3.1805x over previous
import functools

import jax
import jax.numpy as jnp
from jax.experimental import pallas as pl

TOP_K = 2
NUM_EXPERTS = 64
HIDDEN = 2048
TOKENS = 8192

TILE_T = 512


def _router_kernel(x_ref, wt_ref, b_ref, scores_ref, idx_ref):
    x = x_ref[...]
    wt = wt_ref[...]
    logits = jnp.dot(x, wt, preferred_element_type=jnp.float32) + b_ref[...]

    e_iota = jax.lax.broadcasted_iota(jnp.int32, logits.shape, 1)
    big = jnp.int32(NUM_EXPERTS)

    m1 = jnp.max(logits, axis=1, keepdims=True)
    i1 = jnp.min(jnp.where(logits == m1, e_iota, big), axis=1, keepdims=True)

    masked = jnp.where(e_iota == i1, -jnp.inf, logits)
    m2 = jnp.max(masked, axis=1, keepdims=True)
    i2 = jnp.min(jnp.where(masked == m2, e_iota, big), axis=1, keepdims=True)

    d = jnp.exp(m2 - m1)
    denom = 1.0 + d
    p1 = 1.0 / denom
    p2 = d / denom

    scores = jnp.where(e_iota == i1, p1, jnp.where(e_iota == i2, p2, 0.0))
    scores_ref[...] = scores
    idx_ref[...] = jnp.concatenate([i1, i2], axis=1)


@jax.jit
def kernel(hidden_states, W, b):
    x = hidden_states.reshape(-1, HIDDEN)
    wt = W.T
    b2 = b.reshape(1, NUM_EXPERTS)
    grid = (TOKENS // TILE_T,)
    scores, idx = pl.pallas_call(
        _router_kernel,
        grid=grid,
        in_specs=[
            pl.BlockSpec((TILE_T, HIDDEN), lambda i: (i, 0)),
            pl.BlockSpec((HIDDEN, NUM_EXPERTS), lambda i: (0, 0)),
            pl.BlockSpec((1, NUM_EXPERTS), lambda i: (0, 0)),
        ],
        out_specs=[
            pl.BlockSpec((TILE_T, NUM_EXPERTS), lambda i: (i, 0)),
            pl.BlockSpec((TILE_T, TOP_K), lambda i: (i, 0)),
        ],
        out_shape=[
            jax.ShapeDtypeStruct((TOKENS, NUM_EXPERTS), jnp.float32),
            jax.ShapeDtypeStruct((TOKENS, TOP_K), jnp.int32),
        ],
    )(x, wt, b2)
    return scores, idx


# speedup vs baseline: 3.6128x; 1.1359x over previous
import functools

import jax
import jax.numpy as jnp
from jax.experimental import pallas as pl

TOP_K = 2
NUM_EXPERTS = 64
HIDDEN = 2048
TOKENS = 8192

TILE_T = 1024


def _router_kernel(x_ref, wt_ref, b_ref, scores_ref, idx_ref):
    x = x_ref[...]
    wt = wt_ref[...]
    logits = jnp.dot(x, wt, preferred_element_type=jnp.float32) + b_ref[...]

    e_iota = jax.lax.broadcasted_iota(jnp.int32, logits.shape, 1)
    big = jnp.int32(NUM_EXPERTS)

    m1 = jnp.max(logits, axis=1, keepdims=True)
    i1 = jnp.min(jnp.where(logits == m1, e_iota, big), axis=1, keepdims=True)

    masked = jnp.where(e_iota == i1, -jnp.inf, logits)
    m2 = jnp.max(masked, axis=1, keepdims=True)
    i2 = jnp.min(jnp.where(masked == m2, e_iota, big), axis=1, keepdims=True)

    d = jnp.exp(m2 - m1)
    denom = 1.0 + d
    p1 = 1.0 / denom
    p2 = d / denom

    scores = jnp.where(e_iota == i1, p1, jnp.where(e_iota == i2, p2, 0.0))
    scores_ref[...] = scores
    idx_ref[...] = jnp.concatenate([i1, i2], axis=1)


@jax.jit
def kernel(hidden_states, W, b):
    x = hidden_states.reshape(-1, HIDDEN)
    wt = W.T
    b2 = b.reshape(1, NUM_EXPERTS)
    grid = (TOKENS // TILE_T,)
    scores, idx = pl.pallas_call(
        _router_kernel,
        grid=grid,
        in_specs=[
            pl.BlockSpec((TILE_T, HIDDEN), lambda i: (i, 0)),
            pl.BlockSpec((HIDDEN, NUM_EXPERTS), lambda i: (0, 0)),
            pl.BlockSpec((1, NUM_EXPERTS), lambda i: (0, 0)),
        ],
        out_specs=[
            pl.BlockSpec((TILE_T, NUM_EXPERTS), lambda i: (i, 0)),
            pl.BlockSpec((TILE_T, TOP_K), lambda i: (i, 0)),
        ],
        out_shape=[
            jax.ShapeDtypeStruct((TOKENS, NUM_EXPERTS), jnp.float32),
            jax.ShapeDtypeStruct((TOKENS, TOP_K), jnp.int32),
        ],
    )(x, wt, b2)
    return scores, idx


# speedup vs baseline: 3.6394x; 1.0074x over previous
import functools

import jax
import jax.numpy as jnp
from jax.experimental import pallas as pl

TOP_K = 2
NUM_EXPERTS = 64
HIDDEN = 2048
TOKENS = 8192

TILE_T = 2048


def _router_kernel(x_ref, wt_ref, b_ref, scores_ref, idx_ref):
    x = x_ref[...]
    wt = wt_ref[...]
    logits = jnp.dot(x, wt, preferred_element_type=jnp.float32) + b_ref[...]

    e_iota = jax.lax.broadcasted_iota(jnp.int32, logits.shape, 1)
    big = jnp.int32(NUM_EXPERTS)

    m1 = jnp.max(logits, axis=1, keepdims=True)
    i1 = jnp.min(jnp.where(logits == m1, e_iota, big), axis=1, keepdims=True)

    masked = jnp.where(e_iota == i1, -jnp.inf, logits)
    m2 = jnp.max(masked, axis=1, keepdims=True)
    i2 = jnp.min(jnp.where(masked == m2, e_iota, big), axis=1, keepdims=True)

    d = jnp.exp(m2 - m1)
    denom = 1.0 + d
    p1 = 1.0 / denom
    p2 = d / denom

    scores = jnp.where(e_iota == i1, p1, jnp.where(e_iota == i2, p2, 0.0))
    scores_ref[...] = scores
    idx_ref[...] = jnp.concatenate([i1, i2], axis=1)


@jax.jit
def kernel(hidden_states, W, b):
    x = hidden_states.reshape(-1, HIDDEN)
    wt = W.T
    b2 = b.reshape(1, NUM_EXPERTS)
    grid = (TOKENS // TILE_T,)
    scores, idx = pl.pallas_call(
        _router_kernel,
        grid=grid,
        in_specs=[
            pl.BlockSpec((TILE_T, HIDDEN), lambda i: (i, 0)),
            pl.BlockSpec((HIDDEN, NUM_EXPERTS), lambda i: (0, 0)),
            pl.BlockSpec((1, NUM_EXPERTS), lambda i: (0, 0)),
        ],
        out_specs=[
            pl.BlockSpec((TILE_T, NUM_EXPERTS), lambda i: (i, 0)),
            pl.BlockSpec((TILE_T, TOP_K), lambda i: (i, 0)),
        ],
        out_shape=[
            jax.ShapeDtypeStruct((TOKENS, NUM_EXPERTS), jnp.float32),
            jax.ShapeDtypeStruct((TOKENS, TOP_K), jnp.int32),
        ],
    )(x, wt, b2)
    return scores, idx
